# baseline (speedup 1.0000x reference)
import functools

import jax
import jax.numpy as jnp
from jax import lax
from jax.experimental import pallas as pl
from jax.experimental.pallas import tpu as pltpu
from jax.experimental.pallas import tpu_sc as plsc

N = 10000
D = 128
E = 320000
ALPHA = 0.1

NC = 2
NS = 16
NW = NC * NS
CHUNK = 128

NCHUNK = 80
TOTC = NW * NCHUNK
E_PAD = TOTC * CHUNK
KROW = 1
NSTEP = NCHUNK // KROW
NP = 10240
ZCH = NP // NS // CHUNK


@functools.lru_cache(maxsize=None)
def _mesh():
    return plsc.VectorSubcoreMesh(core_axis_name="c", subcore_axis_name="s",
                                  num_cores=NC, num_subcores=NS)


def _deg_body(dst_hbm, zerosd_hbm, onesd_hbm, out_hbm, dst_v, ones_v, part_s):
    c = lax.axis_index("c")
    s = lax.axis_index("s")
    wid = s * NC + c
    for k in range(ZCH):
        pltpu.sync_copy(zerosd_hbm, part_s.at[pl.ds((s * ZCH + k) * CHUNK, CHUNK)])
    pltpu.sync_copy(onesd_hbm, ones_v)
    pltpu.sync_copy(dst_hbm.at[pl.ds(wid * NCHUNK, NCHUNK)], dst_v)
    plsc.subcore_barrier()

    def body(i, carry):
        pltpu.sync_copy(ones_v, part_s.at[dst_v.at[i]], add=True)
        return carry

    lax.fori_loop(0, NCHUNK, body, 0)
    plsc.subcore_barrier()
    for k in range(ZCH):
        r0 = (s * ZCH + k) * CHUNK
        pltpu.sync_copy(part_s.at[pl.ds(r0, CHUNK)], out_hbm.at[c, pl.ds(r0, CHUNK)])


@functools.lru_cache(maxsize=None)
def _deg_kernel():
    return pl.kernel(
        _deg_body,
        out_type=jax.ShapeDtypeStruct((NC, NP, D), jnp.float32),
        mesh=_mesh(),
        scratch_types=[
            pltpu.VMEM((NCHUNK, CHUNK), jnp.int32),
            pltpu.VMEM((CHUNK, D), jnp.float32),
            pltpu.VMEM_SHARED((NP, D), jnp.float32),
        ],
    )


def _deg_call(dst3, zerosd, onesd):
    return _deg_kernel()(dst3, zerosd, onesd)


def _scatter_body(y_hbm, src_hbm, dst_hbm, zerosd_hbm, out_hbm,
                  isrc_v, idst_v, rows_v, part_s, sem):
    c = lax.axis_index("c")
    s = lax.axis_index("s")
    for k in range(ZCH):
        pltpu.sync_copy(zerosd_hbm, part_s.at[pl.ds((s * ZCH + k) * CHUNK, CHUNK)])
    plsc.subcore_barrier()
    wid = s * NC + c

    def body(i, carry):
        base = (wid * NCHUNK + i * KROW) * CHUNK
        pltpu.sync_copy(src_hbm.at[pl.ds(base, KROW * CHUNK)], isrc_v)
        pltpu.async_copy(y_hbm.at[isrc_v], rows_v, sem).wait()
        pltpu.sync_copy(dst_hbm.at[pl.ds(base, KROW * CHUNK)], idst_v)
        pltpu.sync_copy(rows_v, part_s.at[idst_v], add=True)
        return carry

    lax.fori_loop(0, NSTEP, body, 0)
    plsc.subcore_barrier()
    for k in range(ZCH):
        r0 = (s * ZCH + k) * CHUNK
        pltpu.sync_copy(part_s.at[pl.ds(r0, CHUNK)], out_hbm.at[c, pl.ds(r0, CHUNK)])


@functools.lru_cache(maxsize=None)
def _scatter_kernel():
    return pl.kernel(
        _scatter_body,
        out_type=jax.ShapeDtypeStruct((NC, NP, D), jnp.float32),
        mesh=_mesh(),
        scratch_types=[
            pltpu.VMEM((KROW * CHUNK,), jnp.int32),
            pltpu.VMEM((KROW * CHUNK,), jnp.int32),
            pltpu.VMEM((KROW * CHUNK, D), jnp.float32),
            pltpu.VMEM_SHARED((NP, D), jnp.float32),
            pltpu.SemaphoreType.DMA,
        ],
    )


def _scatter_call(y, src3, dst3, zerosd):
    return _scatter_kernel()(y, src3, dst3, zerosd)


BLK = 1000


def _prep_kernel(dp_ref, x_ref, dinv_ref, y_ref):
    dp = dp_ref[...]
    deg = dp[0, :, :1] + dp[1, :, :1] + 1.0
    dinv = lax.rsqrt(deg)
    db = jnp.broadcast_to(dinv, (BLK, D))
    dinv_ref[...] = db
    y_ref[...] = x_ref[...] * db


def _prep_call(deg_parts, x):
    return pl.pallas_call(
        _prep_kernel,
        grid=(N // BLK,),
        in_specs=[
            pl.BlockSpec((NC, BLK, D), lambda i: (0, i, 0)),
            pl.BlockSpec((BLK, D), lambda i: (i, 0)),
        ],
        out_specs=[
            pl.BlockSpec((BLK, D), lambda i: (i, 0)),
            pl.BlockSpec((BLK, D), lambda i: (i, 0)),
        ],
        out_shape=[
            jax.ShapeDtypeStruct((N, D), jnp.float32),
            jax.ShapeDtypeStruct((N, D), jnp.float32),
        ],
    )(deg_parts, x)


def _layer_kernel(sp_ref, y_ref, xin_ref, dinv_ref, w_ref, h_ref, ynext_ref):
    sp = sp_ref[...]
    agg = dinv_ref[...] * (sp[0] + sp[1] + y_ref[...])
    t = (1.0 - ALPHA) * agg + ALPHA * xin_ref[...]
    h = jnp.dot(t, w_ref[...], preferred_element_type=jnp.float32)
    h_ref[...] = h
    ynext_ref[...] = h * dinv_ref[...]


def _layer_call(s_parts, y, x_in, dinv_b, w):
    return pl.pallas_call(
        _layer_kernel,
        grid=(N // BLK,),
        in_specs=[
            pl.BlockSpec((NC, BLK, D), lambda i: (0, i, 0)),
            pl.BlockSpec((BLK, D), lambda i: (i, 0)),
            pl.BlockSpec((BLK, D), lambda i: (i, 0)),
            pl.BlockSpec((BLK, D), lambda i: (i, 0)),
            pl.BlockSpec((D, D), lambda i: (0, 0)),
        ],
        out_specs=[
            pl.BlockSpec((BLK, D), lambda i: (i, 0)),
            pl.BlockSpec((BLK, D), lambda i: (i, 0)),
        ],
        out_shape=[
            jax.ShapeDtypeStruct((N, D), jnp.float32),
            jax.ShapeDtypeStruct((N, D), jnp.float32),
        ],
    )(s_parts, y, x_in, dinv_b, w)


def kernel(x, edge_index, W1, W2):
    src = edge_index[0]
    dst = edge_index[1]
    pad = E_PAD - E
    src_p = jnp.concatenate([src, jnp.zeros((pad,), jnp.int32)])
    dst_p = jnp.concatenate([dst, jnp.full((pad,), N, jnp.int32)])
    dst3 = dst_p.reshape(TOTC, CHUNK)
    zerosd = jnp.zeros((CHUNK, D), jnp.float32)
    onesd = jnp.ones((CHUNK, D), jnp.float32)

    deg_parts = _deg_call(dst3, zerosd, onesd)
    dinv_b, y1 = _prep_call(deg_parts, x)
    s1 = _scatter_call(y1, src_p, dst_p, zerosd)
    h1, y2 = _layer_call(s1, y1, x, dinv_b, W1)
    s2 = _scatter_call(y2, src_p, dst_p, zerosd)
    h2, _ = _layer_call(s2, y2, h1, dinv_b, W2)
    return h2


# speedup vs baseline: 1.3138x; 1.3138x over previous
import functools

import jax
import jax.numpy as jnp
from jax import lax
from jax.experimental import pallas as pl
from jax.experimental.pallas import tpu as pltpu
from jax.experimental.pallas import tpu_sc as plsc

N = 10000
D = 128
E = 320000
ALPHA = 0.1

NC = 2
NS = 16
NW = NC * NS
CHUNK = 128

E_PAD = ((E + NW * CHUNK - 1) // (NW * CHUNK)) * (NW * CHUNK)
EW = E_PAD // NW
NCHUNK = EW // CHUNK
NP = 10240
ZCH = NP // NS // CHUNK


@functools.lru_cache(maxsize=None)
def _mesh():
    return plsc.VectorSubcoreMesh(core_axis_name="c", subcore_axis_name="s",
                                  num_cores=NC, num_subcores=NS)


def _deg_body(dst_hbm, zerosd_hbm, onesd_hbm, out_hbm, idx_v, ones_v, part_s):
    c = lax.axis_index("c")
    s = lax.axis_index("s")
    wid = s * NC + c
    for k in range(ZCH):
        pltpu.sync_copy(zerosd_hbm, part_s.at[pl.ds((s * ZCH + k) * CHUNK, CHUNK)])
    pltpu.sync_copy(onesd_hbm, ones_v)
    plsc.subcore_barrier()

    def body(i, carry):
        base = wid * EW + i * CHUNK
        pltpu.sync_copy(dst_hbm.at[pl.ds(base, CHUNK)], idx_v)
        pltpu.sync_copy(ones_v, part_s.at[idx_v], add=True)
        return carry

    lax.fori_loop(0, NCHUNK, body, 0)
    plsc.subcore_barrier()
    for k in range(ZCH):
        r0 = (s * ZCH + k) * CHUNK
        pltpu.sync_copy(part_s.at[pl.ds(r0, CHUNK)], out_hbm.at[c, pl.ds(r0, CHUNK)])


@functools.lru_cache(maxsize=None)
def _deg_kernel():
    return pl.kernel(
        _deg_body,
        out_type=jax.ShapeDtypeStruct((NC, NP, D), jnp.float32),
        mesh=_mesh(),
        scratch_types=[
            pltpu.VMEM((CHUNK,), jnp.int32),
            pltpu.VMEM((CHUNK, D), jnp.float32),
            pltpu.VMEM_SHARED((NP, D), jnp.float32),
        ],
    )


def _deg_call(dst_p, zerosd, onesd):
    return _deg_kernel()(dst_p, zerosd, onesd)


def _scatter_body(y_hbm, src_hbm, dst_hbm, zerosd_hbm, out_hbm,
                  isrc_v, idst_v, rows_v, part_s, sem):
    c = lax.axis_index("c")
    s = lax.axis_index("s")
    wid = s * NC + c
    for k in range(ZCH):
        pltpu.sync_copy(zerosd_hbm, part_s.at[pl.ds((s * ZCH + k) * CHUNK, CHUNK)])
    plsc.subcore_barrier()

    def body(i, carry):
        base = wid * EW + i * CHUNK
        pltpu.sync_copy(src_hbm.at[pl.ds(base, CHUNK)], isrc_v)
        pltpu.async_copy(y_hbm.at[isrc_v], rows_v, sem).wait()
        pltpu.sync_copy(dst_hbm.at[pl.ds(base, CHUNK)], idst_v)
        pltpu.sync_copy(rows_v, part_s.at[idst_v], add=True)
        return carry

    lax.fori_loop(0, NCHUNK, body, 0)
    plsc.subcore_barrier()
    for k in range(ZCH):
        r0 = (s * ZCH + k) * CHUNK
        pltpu.sync_copy(part_s.at[pl.ds(r0, CHUNK)], out_hbm.at[c, pl.ds(r0, CHUNK)])


@functools.lru_cache(maxsize=None)
def _scatter_kernel():
    return pl.kernel(
        _scatter_body,
        out_type=jax.ShapeDtypeStruct((NC, NP, D), jnp.float32),
        mesh=_mesh(),
        scratch_types=[
            pltpu.VMEM((CHUNK,), jnp.int32),
            pltpu.VMEM((CHUNK,), jnp.int32),
            pltpu.VMEM((CHUNK, D), jnp.float32),
            pltpu.VMEM_SHARED((NP, D), jnp.float32),
            pltpu.SemaphoreType.DMA,
        ],
    )


def _scatter_call(y, src_p, dst_p, zerosd):
    return _scatter_kernel()(y, src_p, dst_p, zerosd)


BLK = 1000


def _prep_kernel(dp_ref, x_ref, dinv_ref, y_ref):
    dp = dp_ref[...]
    deg = dp[0, :, :1] + dp[1, :, :1] + 1.0
    dinv = lax.rsqrt(deg)
    db = jnp.broadcast_to(dinv, (BLK, D))
    dinv_ref[...] = db
    y_ref[...] = x_ref[...] * db


def _prep_call(deg_parts, x):
    return pl.pallas_call(
        _prep_kernel,
        grid=(N // BLK,),
        in_specs=[
            pl.BlockSpec((NC, BLK, D), lambda i: (0, i, 0)),
            pl.BlockSpec((BLK, D), lambda i: (i, 0)),
        ],
        out_specs=[
            pl.BlockSpec((BLK, D), lambda i: (i, 0)),
            pl.BlockSpec((BLK, D), lambda i: (i, 0)),
        ],
        out_shape=[
            jax.ShapeDtypeStruct((N, D), jnp.float32),
            jax.ShapeDtypeStruct((N, D), jnp.float32),
        ],
    )(deg_parts, x)


def _layer_kernel(sp_ref, y_ref, xin_ref, dinv_ref, w_ref, h_ref, ynext_ref):
    sp = sp_ref[...]
    agg = dinv_ref[...] * (sp[0] + sp[1] + y_ref[...])
    t = (1.0 - ALPHA) * agg + ALPHA * xin_ref[...]
    h = jnp.dot(t, w_ref[...], preferred_element_type=jnp.float32)
    h_ref[...] = h
    ynext_ref[...] = h * dinv_ref[...]


def _layer_call(s_parts, y, x_in, dinv_b, w):
    return pl.pallas_call(
        _layer_kernel,
        grid=(N // BLK,),
        in_specs=[
            pl.BlockSpec((NC, BLK, D), lambda i: (0, i, 0)),
            pl.BlockSpec((BLK, D), lambda i: (i, 0)),
            pl.BlockSpec((BLK, D), lambda i: (i, 0)),
            pl.BlockSpec((BLK, D), lambda i: (i, 0)),
            pl.BlockSpec((D, D), lambda i: (0, 0)),
        ],
        out_specs=[
            pl.BlockSpec((BLK, D), lambda i: (i, 0)),
            pl.BlockSpec((BLK, D), lambda i: (i, 0)),
        ],
        out_shape=[
            jax.ShapeDtypeStruct((N, D), jnp.float32),
            jax.ShapeDtypeStruct((N, D), jnp.float32),
        ],
    )(s_parts, y, x_in, dinv_b, w)


def kernel(x, edge_index, W1, W2):
    src = edge_index[0]
    dst = edge_index[1]
    pad = E_PAD - E
    src_p = jnp.concatenate([src, jnp.zeros((pad,), jnp.int32)])
    dst_p = jnp.concatenate([dst, jnp.full((pad,), N, jnp.int32)])
    zerosd = jnp.zeros((CHUNK, D), jnp.float32)
    onesd = jnp.ones((CHUNK, D), jnp.float32)

    deg_parts = _deg_call(dst_p, zerosd, onesd)
    dinv_b, y1 = _prep_call(deg_parts, x)
    s1 = _scatter_call(y1, src_p, dst_p, zerosd)
    h1, y2 = _layer_call(s1, y1, x, dinv_b, W1)
    s2 = _scatter_call(y2, src_p, dst_p, zerosd)
    h2, _ = _layer_call(s2, y2, h1, dinv_b, W2)
    return h2
